# speedup vs baseline: 5.0444x; 5.0444x over previous
import jax
import jax.numpy as jnp
from jax.experimental import pallas as pl

_BS = 512


def _body(table_ref, out_ref):
    rows = table_ref[...]
    out_ref[...] = jnp.broadcast_to(rows[None], out_ref.shape)


def kernel(x, position_embeddings):
    batch = x.shape[0]
    seq_len = x.shape[1]
    n_rows, dim = position_embeddings.shape
    grid = (seq_len // _BS,)
    out = pl.pallas_call(
        _body,
        grid=grid,
        in_specs=[pl.BlockSpec((_BS, dim), lambda i: (i, 0))],
        out_specs=pl.BlockSpec((batch, _BS, dim), lambda i: (0, i, 0)),
        out_shape=jax.ShapeDtypeStruct((batch, seq_len, dim), position_embeddings.dtype),
    )(position_embeddings)
    return out
